# Initial kernel scaffold; baseline (speedup 1.0000x reference)
import jax
import jax.numpy as jnp
from jax.experimental import pallas as pl


def kernel(input_ids, token_table, pos_table):
    raise NotImplementedError("write your pallas kernel here")


# speedup vs baseline: 1.0369x; 1.0369x over previous
import functools

import jax
import jax.numpy as jnp
from jax import lax
from jax.experimental import pallas as pl
from jax.experimental.pallas import tpu as pltpu
from jax.experimental.pallas import tpu_sc as plsc

D = 512
NC = 2
NS = 16
NW = NC * NS
L = 16


@functools.cache
def _make_sc_lookup(B, T):
    N = B * T
    NPW = N // NW
    C = 64
    NCH = NPW // C

    mesh = plsc.VectorSubcoreMesh(core_axis_name="c", subcore_axis_name="s",
                                  num_cores=NC, num_subcores=NS)

    @functools.partial(
        pl.kernel,
        out_type=jax.ShapeDtypeStruct((N, D), jnp.float32),
        mesh=mesh,
        scratch_types=[
            pltpu.VMEM((NPW,), jnp.int32),
            pltpu.VMEM((C, D), jnp.float32),
            pltpu.VMEM((C, D), jnp.float32),
            pltpu.SemaphoreType.DMA,
            pltpu.SemaphoreType.DMA,
        ],
    )
    def lookup(ids_hbm, tok_hbm, pos_hbm, out_hbm, idx_v, acc_v, rows_v,
               gsem, psem):
        wid = lax.axis_index("s") * NC + lax.axis_index("c")
        base = wid * NPW
        t0 = lax.rem(base, T)
        pltpu.sync_copy(ids_hbm.at[pl.ds(base, NPW)], idx_v)
        for ch in range(NCH):
            off = ch * C
            pcopy = pltpu.async_copy(pos_hbm.at[pl.ds(t0 + off, C)], acc_v, psem)
            gcopy = pltpu.async_copy(tok_hbm.at[idx_v.at[pl.ds(off, C)]],
                                     rows_v, gsem)
            pcopy.wait()
            gcopy.wait()

            def row_add(r, carry):
                for j in range(D // L):
                    plsc.addupdate(acc_v.at[r, pl.ds(j * L, L)],
                                   rows_v[r, pl.ds(j * L, L)])
                return carry

            lax.fori_loop(0, C, row_add, 0)
            pltpu.sync_copy(acc_v, out_hbm.at[pl.ds(base + off, C)])

    return lookup


def kernel(input_ids, token_table, pos_table):
    B, T = input_ids.shape
    ids = input_ids.reshape(-1).astype(jnp.int32)
    out = _make_sc_lookup(B, T)(ids, token_table, pos_table)
    return out.reshape(B, T, D)


# speedup vs baseline: 1.1761x; 1.1342x over previous
import functools

import jax
import jax.numpy as jnp
from jax import lax
from jax.experimental import pallas as pl
from jax.experimental.pallas import tpu as pltpu
from jax.experimental.pallas import tpu_sc as plsc

D = 512
NC = 2
NS = 16
NW = NC * NS
L = 16


@functools.cache
def _make_sc_lookup(B, T):
    N = B * T
    NPW = N // NW
    C = 32
    NCH = NPW // C
    NB = 2

    mesh = plsc.VectorSubcoreMesh(core_axis_name="c", subcore_axis_name="s",
                                  num_cores=NC, num_subcores=NS)

    @functools.partial(
        pl.kernel,
        out_type=jax.ShapeDtypeStruct((N, D), jnp.float32),
        mesh=mesh,
        scratch_types=[
            pltpu.VMEM((NPW,), jnp.int32),
            [pltpu.VMEM((C, D), jnp.float32) for _ in range(NB)],
            [pltpu.VMEM((C, D), jnp.float32) for _ in range(NB)],
            [pltpu.SemaphoreType.DMA for _ in range(NB)],
            [pltpu.SemaphoreType.DMA for _ in range(NB)],
            [pltpu.SemaphoreType.DMA for _ in range(NB)],
        ],
    )
    def lookup(ids_hbm, tok_hbm, pos_hbm, out_hbm, idx_v, acc_v, rows_v,
               gsem, psem, wsem):
        wid = lax.axis_index("s") * NC + lax.axis_index("c")
        base = wid * NPW
        t0 = lax.rem(base, T)
        pltpu.sync_copy(ids_hbm.at[pl.ds(base, NPW)], idx_v)

        def prefetch(ch, b):
            off = ch * C
            pltpu.async_copy(pos_hbm.at[pl.ds(t0 + off, C)], acc_v[b], psem[b])
            pltpu.async_copy(tok_hbm.at[idx_v.at[pl.ds(off, C)]],
                             rows_v[b], gsem[b])

        prefetch(0, 0)
        for ch in range(NCH):
            b = ch % NB
            nxt = (ch + 1) % NB
            if ch + 1 < NCH:
                if ch + 1 >= NB:
                    pltpu.make_async_copy(acc_v[nxt],
                                          out_hbm.at[pl.ds(0, C)],
                                          wsem[nxt]).wait()
                prefetch(ch + 1, nxt)
            pltpu.make_async_copy(pos_hbm.at[pl.ds(0, C)], acc_v[b],
                                  psem[b]).wait()
            pltpu.make_async_copy(tok_hbm.at[pl.ds(0, C)], rows_v[b],
                                  gsem[b]).wait()

            def row_add(r, carry, b=b):
                for j in range(D // L):
                    plsc.addupdate(acc_v[b].at[r, pl.ds(j * L, L)],
                                   rows_v[b][r, pl.ds(j * L, L)])
                return carry

            lax.fori_loop(0, C, row_add, 0)
            pltpu.async_copy(acc_v[b], out_hbm.at[pl.ds(base + ch * C, C)],
                             wsem[b])
        for ch in range(max(0, NCH - NB), NCH):
            b = ch % NB
            pltpu.make_async_copy(acc_v[b], out_hbm.at[pl.ds(0, C)],
                                  wsem[b]).wait()

    return lookup


def kernel(input_ids, token_table, pos_table):
    B, T = input_ids.shape
    ids = input_ids.reshape(-1).astype(jnp.int32)
    out = _make_sc_lookup(B, T)(ids, token_table, pos_table)
    return out.reshape(B, T, D)


# speedup vs baseline: 1.3932x; 1.1846x over previous
import functools

import jax
import jax.numpy as jnp
from jax import lax
from jax.experimental import pallas as pl
from jax.experimental.pallas import tpu as pltpu
from jax.experimental.pallas import tpu_sc as plsc

D = 512
NC = 2
NS = 16
NW = NC * NS
L = 16


@functools.cache
def _make_sc_lookup(B, T):
    N = B * T
    TW = T // NW
    NPW = B * TW
    NB = 2

    mesh = plsc.VectorSubcoreMesh(core_axis_name="c", subcore_axis_name="s",
                                  num_cores=NC, num_subcores=NS)

    @functools.partial(
        pl.kernel,
        out_type=jax.ShapeDtypeStruct((N, D), jnp.float32),
        mesh=mesh,
        scratch_types=[
            pltpu.VMEM((NPW,), jnp.int32),
            pltpu.VMEM((TW, D), jnp.float32),
            [pltpu.VMEM((TW, D), jnp.float32) for _ in range(NB)],
            pltpu.SemaphoreType.DMA,
            pltpu.SemaphoreType.DMA,
            [pltpu.SemaphoreType.DMA for _ in range(NB)],
            [pltpu.SemaphoreType.DMA for _ in range(NB)],
        ],
    )
    def lookup(ids_hbm, tok_hbm, pos_hbm, out_hbm, idx_v, pos_v, rows_v,
               psem, isem, gsem, wsem):
        wid = lax.axis_index("s") * NC + lax.axis_index("c")
        t0 = wid * TW
        pltpu.async_copy(pos_hbm.at[pl.ds(t0, TW)], pos_v, psem)
        for b in range(B):
            pltpu.async_copy(ids_hbm.at[pl.ds(b * T + t0, TW)],
                             idx_v.at[pl.ds(b * TW, TW)], isem)
        for b in range(B):
            pltpu.make_async_copy(ids_hbm.at[pl.ds(0, TW)],
                                  idx_v.at[pl.ds(0, TW)], isem).wait()

        def prefetch(b, buf):
            pltpu.async_copy(tok_hbm.at[idx_v.at[pl.ds(b * TW, TW)]],
                             rows_v[buf], gsem[buf])

        prefetch(0, 0)
        pltpu.make_async_copy(pos_hbm.at[pl.ds(0, TW)], pos_v, psem).wait()
        for b in range(B):
            buf = b % NB
            nxt = (b + 1) % NB
            if b + 1 < B:
                if b + 1 >= NB:
                    pltpu.make_async_copy(rows_v[nxt],
                                          out_hbm.at[pl.ds(0, TW)],
                                          wsem[nxt]).wait()
                prefetch(b + 1, nxt)
            pltpu.make_async_copy(tok_hbm.at[pl.ds(0, TW)], rows_v[buf],
                                  gsem[buf]).wait()

            def row_add(r, carry, buf=buf):
                for j in range(D // L):
                    plsc.addupdate(rows_v[buf].at[r, pl.ds(j * L, L)],
                                   pos_v[r, pl.ds(j * L, L)])
                return carry

            lax.fori_loop(0, TW, row_add, 0)
            pltpu.async_copy(rows_v[buf], out_hbm.at[pl.ds(b * T + t0, TW)],
                             wsem[buf])
        for b in range(max(0, B - NB), B):
            buf = b % NB
            pltpu.make_async_copy(rows_v[buf], out_hbm.at[pl.ds(0, TW)],
                                  wsem[buf]).wait()

    return lookup


def kernel(input_ids, token_table, pos_table):
    B, T = input_ids.shape
    ids = input_ids.reshape(-1).astype(jnp.int32)
    out = _make_sc_lookup(B, T)(ids, token_table, pos_table)
    return out.reshape(B, T, D)


# speedup vs baseline: 1.3940x; 1.0006x over previous
import functools

import jax
import jax.numpy as jnp
from jax import lax
from jax.experimental import pallas as pl
from jax.experimental.pallas import tpu as pltpu
from jax.experimental.pallas import tpu_sc as plsc

D = 512
NC = 2
NS = 16
NW = NC * NS
L = 16


@functools.cache
def _make_sc_lookup(B, T):
    N = B * T
    TW = T // NW
    NPW = B * TW
    NB = 2

    mesh = plsc.VectorSubcoreMesh(core_axis_name="c", subcore_axis_name="s",
                                  num_cores=NC, num_subcores=NS)

    @functools.partial(
        pl.kernel,
        out_type=jax.ShapeDtypeStruct((N, D), jnp.float32),
        mesh=mesh,
        scratch_types=[
            pltpu.VMEM((NPW,), jnp.int32),
            pltpu.VMEM((TW, D), jnp.float32),
            [pltpu.VMEM((TW, D), jnp.float32) for _ in range(NB)],
            pltpu.SemaphoreType.DMA,
            pltpu.SemaphoreType.DMA,
            [pltpu.SemaphoreType.DMA for _ in range(NB)],
            [pltpu.SemaphoreType.DMA for _ in range(NB)],
        ],
    )
    def lookup(ids_hbm, tok_hbm, pos_hbm, out_hbm, idx_v, pos_v, rows_v,
               psem, isem, gsem, wsem):
        wid = lax.axis_index("s") * NC + lax.axis_index("c")
        t0 = wid * TW
        pltpu.async_copy(pos_hbm.at[pl.ds(t0, TW)], pos_v, psem)
        for b in range(B):
            pltpu.async_copy(ids_hbm.at[b, pl.ds(t0, TW)],
                             idx_v.at[pl.ds(b * TW, TW)], isem)
        for b in range(B):
            pltpu.make_async_copy(ids_hbm.at[0, pl.ds(0, TW)],
                                  idx_v.at[pl.ds(0, TW)], isem).wait()

        def prefetch(b, buf):
            pltpu.async_copy(tok_hbm.at[idx_v.at[pl.ds(b * TW, TW)]],
                             rows_v[buf], gsem[buf])

        prefetch(0, 0)
        pltpu.make_async_copy(pos_hbm.at[pl.ds(0, TW)], pos_v, psem).wait()
        for b in range(B):
            buf = b % NB
            nxt = (b + 1) % NB
            if b + 1 < B:
                if b + 1 >= NB:
                    pltpu.make_async_copy(rows_v[nxt],
                                          out_hbm.at[pl.ds(0, TW)],
                                          wsem[nxt]).wait()
                prefetch(b + 1, nxt)
            pltpu.make_async_copy(tok_hbm.at[pl.ds(0, TW)], rows_v[buf],
                                  gsem[buf]).wait()

            def row_add(r, carry, buf=buf):
                for j in range(D // L):
                    plsc.addupdate(rows_v[buf].at[r, pl.ds(j * L, L)],
                                   pos_v[r, pl.ds(j * L, L)])
                return carry

            lax.fori_loop(0, TW, row_add, 0)
            pltpu.async_copy(rows_v[buf], out_hbm.at[pl.ds(b * T + t0, TW)],
                             wsem[buf])
        for b in range(max(0, B - NB), B):
            buf = b % NB
            pltpu.make_async_copy(rows_v[buf], out_hbm.at[pl.ds(0, TW)],
                                  wsem[buf]).wait()

    return lookup


def kernel(input_ids, token_table, pos_table):
    B, T = input_ids.shape
    ids = input_ids.astype(jnp.int32)
    out = _make_sc_lookup(B, T)(ids, token_table, pos_table)
    return out.reshape(B, T, D)
